# baseline (speedup 1.0000x reference)
import jax
import jax.numpy as jnp
from jax.experimental import pallas as pl

STRENGTH = 0.9
THRESHOLD = 0.5

ROWS, COLS = 16384, 1024
BLOCK_ROWS = 4096
BLOCK_COLS = 512


def _body(z_ref, o_ref):
    j = pl.program_id(1)
    zb = z_ref[...]

    @pl.when(j == 0)
    def _():
        wet = jax.nn.sigmoid((zb[:, 0:1] - THRESHOLD) * 10.0) * STRENGTH
        lane = jax.lax.broadcasted_iota(jnp.int32, zb.shape, 1)
        o_ref[...] = jnp.where(lane == 1, wet, zb)

    @pl.when(j != 0)
    def _():
        o_ref[...] = zb


def kernel(z):
    grid = (ROWS // BLOCK_ROWS, COLS // BLOCK_COLS)
    return pl.pallas_call(
        _body,
        grid=grid,
        in_specs=[pl.BlockSpec((BLOCK_ROWS, BLOCK_COLS), lambda i, j: (i, j))],
        out_specs=pl.BlockSpec((BLOCK_ROWS, BLOCK_COLS), lambda i, j: (i, j)),
        out_shape=jax.ShapeDtypeStruct((ROWS, COLS), jnp.float32),
    )(z)


# speedup vs baseline: 1.0341x; 1.0341x over previous
import jax
import jax.numpy as jnp
from jax.experimental import pallas as pl

STRENGTH = 0.9
THRESHOLD = 0.5

ROWS, COLS = 16384, 1024
BLOCK_ROWS = 2048


def _body(z_ref, o_ref):
    zb = z_ref[...]
    wet = jax.nn.sigmoid((zb[:, 0:1] - THRESHOLD) * 10.0) * STRENGTH
    lane = jax.lax.broadcasted_iota(jnp.int32, zb.shape, 1)
    o_ref[...] = jnp.where(lane == 1, wet, zb)


def kernel(z):
    grid = (ROWS // BLOCK_ROWS,)
    return pl.pallas_call(
        _body,
        grid=grid,
        in_specs=[pl.BlockSpec((BLOCK_ROWS, COLS), lambda i: (i, 0))],
        out_specs=pl.BlockSpec((BLOCK_ROWS, COLS), lambda i: (i, 0)),
        out_shape=jax.ShapeDtypeStruct((ROWS, COLS), jnp.float32),
    )(z)
